# baseline (speedup 1.0000x reference)
import functools

import jax
import jax.numpy as jnp
import numpy as np
from jax import lax
from jax.experimental import pallas as pl
from jax.experimental.pallas import tpu as pltpu
from jax.experimental.pallas import tpu_sc as plsc

NUM_BUCKETS = 32
MAX_DISTANCE = 128
N_HEADS = 16
SEQ = 2048
NSHIFT = 16
TABW = 4096
PADW = 4224
BW = 3072
BUILD_W = 2944
ROWS_PER_SUBCORE = SEQ // 2
NGROUPS = 16


def _tables_tc_kernel(bias_ref, d_ref, bid_ref, out_ref):
    d = d_ref[0, 0]
    bid = bid_ref[0, 0]
    pos = lax.broadcasted_iota(jnp.int32, (NUM_BUCKETS, PADW), 1)
    rel = pos - (SEQ - 1) + d
    half = NUM_BUCKETS // 2
    max_exact = half // 2
    rel_buckets = (rel > 0).astype(jnp.int32) * half * bid
    a = jnp.abs(rel)
    is_small = a < max_exact
    rp_large = max_exact + (
        jnp.log(a.astype(jnp.float32) / max_exact)
        / np.log(MAX_DISTANCE / max_exact)
        * (half - max_exact)
    ).astype(jnp.int32)
    rp_large = jnp.minimum(rp_large, jnp.full_like(rp_large, half - 1))
    bucket = jnp.where(is_small, a, rp_large) + rel_buckets
    onehot = (bucket == lax.broadcasted_iota(jnp.int32, (NUM_BUCKETS, PADW), 0))
    table = lax.dot_general(
        bias_ref[...],
        onehot.astype(jnp.float32),
        (((0,), (0,)), ((), ())),
        preferred_element_type=jnp.float32,
        precision=lax.Precision.HIGHEST,
    )
    for s in range(NSHIFT):
        out_ref[:, s, :] = table[:, s : s + TABW]


def _compute_tables(bias, d_arr, bid_arr):
    return pl.pallas_call(
        _tables_tc_kernel,
        out_shape=jax.ShapeDtypeStruct((N_HEADS, NSHIFT, TABW), jnp.float32),
        in_specs=[
            pl.BlockSpec((NUM_BUCKETS, N_HEADS), lambda: (0, 0)),
            pl.BlockSpec(memory_space=pltpu.SMEM),
            pl.BlockSpec(memory_space=pltpu.SMEM),
        ],
        out_specs=pl.BlockSpec((N_HEADS, NSHIFT, TABW), lambda: (0, 0, 0)),
    )(bias, d_arr, bid_arr)


def _sc_expand_body(tables_hbm, out_hbm, tab16_v, b_v, sem0, sem1):
    wid = lax.axis_index("s") * 2 + lax.axis_index("c")
    head = wid // 2
    half = wid % 2
    pltpu.sync_copy(
        tables_hbm.at[pl.ds(pl.multiple_of(head * (NSHIFT * TABW), 8), NSHIFT * TABW)],
        tab16_v,
    )
    s0 = (SEQ - 1) - half * ROWS_PER_SUBCORE
    i_base = half * ROWS_PER_SUBCORE

    def build_and_fire(g, slot, sem):
        a0 = s0 - 8 * g - 896
        fbases = []
        for r in range(8):
            ar = a0 - r
            sh = lax.bitwise_and(ar, NSHIFT - 1)
            fbases.append(sh * TABW + (ar - sh))

        @plsc.parallel_loop(0, BUILD_W // 128)
        def build_body(u):
            col = u * 128
            for r in range(8):
                for j in range(8):
                    off = col + 16 * j
                    v = tab16_v[pl.ds(pl.multiple_of(fbases[r] + off, 16), 16)]
                    b_v[slot, r, pl.ds(pl.multiple_of(off, 16), 16)] = v
        for n in range(8):
            i0 = i_base + 8 * g + 128 * n
            pltpu.make_async_copy(
                b_v.at[slot, :, pl.ds(pl.multiple_of(896 - 128 * n, 128), SEQ)],
                out_hbm.at[0, head, pl.ds(i0, 8), :],
                sem,
            ).start()

    def drain_group(slot, sem):
        for _ in range(8):
            pltpu.make_async_copy(
                b_v.at[slot, :, pl.ds(0, SEQ)],
                out_hbm.at[0, head, pl.ds(i_base, 8), :],
                sem,
            ).wait()

    def pair_body(gp, carry):
        @pl.when(gp > 0)
        def _():
            drain_group(0, sem0)

        build_and_fire(2 * gp, 0, sem0)

        @pl.when(gp > 0)
        def _():
            drain_group(1, sem1)

        build_and_fire(2 * gp + 1, 1, sem1)
        return carry

    lax.fori_loop(0, NGROUPS // 2, pair_body, 0)
    drain_group(0, sem0)
    drain_group(1, sem1)


@functools.cache
def _sc_expand():
    return pl.kernel(
        _sc_expand_body,
        out_type=jax.ShapeDtypeStruct((1, N_HEADS, SEQ, SEQ), jnp.float32),
        mesh=plsc.VectorSubcoreMesh(core_axis_name="c", subcore_axis_name="s"),
        scratch_types=[
            pltpu.VMEM((NSHIFT * TABW,), jnp.float32),
            pltpu.VMEM((2, 8, BW), jnp.float32),
            pltpu.SemaphoreType.DMA,
            pltpu.SemaphoreType.DMA,
        ],
    )


def kernel(relative_attention_bias, q_len, k_len, bidirectional):
    d = jnp.asarray(k_len, jnp.int32) - jnp.asarray(q_len, jnp.int32)
    d_arr = jnp.reshape(d, (1, 1))
    bid_arr = jnp.reshape(jnp.asarray(bidirectional, jnp.int32), (1, 1))
    tables = _compute_tables(relative_attention_bias, d_arr, bid_arr)
    return _sc_expand()(jnp.reshape(tables, (N_HEADS * NSHIFT * TABW,)))


# speedup vs baseline: 1.0120x; 1.0120x over previous
import functools

import jax
import jax.numpy as jnp
import numpy as np
from jax import lax
from jax.experimental import pallas as pl
from jax.experimental.pallas import tpu as pltpu
from jax.experimental.pallas import tpu_sc as plsc

NUM_BUCKETS = 32
MAX_DISTANCE = 128
N_HEADS = 16
SEQ = 2048
NSHIFT = 16
TABW = 4096
PADW = 4224
BW = 3072
BUILD_W = 2944
ROWS_PER_SUBCORE = SEQ // 2
NGROUPS = 16


def _tables_tc_kernel(bias_t_ref, d_ref, bid_ref, out_ref):
    d = d_ref[0, 0]
    bid = bid_ref[0, 0]
    pos = lax.broadcasted_iota(jnp.int32, (NUM_BUCKETS, PADW), 1)
    rel = pos - (SEQ - 1) + d
    half = NUM_BUCKETS // 2
    max_exact = half // 2
    rel_buckets = (rel > 0).astype(jnp.int32) * half * bid
    a = jnp.abs(rel)
    is_small = a < max_exact
    rp_large = max_exact + (
        jnp.log(a.astype(jnp.float32) / max_exact)
        / np.log(MAX_DISTANCE / max_exact)
        * (half - max_exact)
    ).astype(jnp.int32)
    rp_large = jnp.minimum(rp_large, jnp.full_like(rp_large, half - 1))
    bucket = jnp.where(is_small, a, rp_large) + rel_buckets
    onehot = (bucket == lax.broadcasted_iota(jnp.int32, (NUM_BUCKETS, PADW), 0))
    table = jnp.dot(
        bias_t_ref[...],
        onehot.astype(jnp.float32),
        preferred_element_type=jnp.float32,
        precision=lax.Precision.HIGHEST,
    )
    for s in range(NSHIFT):
        out_ref[:, s, :] = table[:, s : s + TABW]


def _compute_tables(bias_t, d_arr, bid_arr):
    return pl.pallas_call(
        _tables_tc_kernel,
        out_shape=jax.ShapeDtypeStruct((N_HEADS, NSHIFT, TABW), jnp.float32),
        in_specs=[
            pl.BlockSpec((N_HEADS, NUM_BUCKETS), lambda: (0, 0)),
            pl.BlockSpec(memory_space=pltpu.SMEM),
            pl.BlockSpec(memory_space=pltpu.SMEM),
        ],
        out_specs=pl.BlockSpec((N_HEADS, NSHIFT, TABW), lambda: (0, 0, 0)),
    )(bias_t, d_arr, bid_arr)


def _sc_expand_body(tables_hbm, out_hbm, tab16_v, b_v, sem0, sem1):
    wid = lax.axis_index("s") * 2 + lax.axis_index("c")
    head = wid // 2
    half = wid % 2
    pltpu.sync_copy(
        tables_hbm.at[pl.ds(pl.multiple_of(head * (NSHIFT * TABW), 8), NSHIFT * TABW)],
        tab16_v,
    )
    s0 = (SEQ - 1) - half * ROWS_PER_SUBCORE
    i_base = half * ROWS_PER_SUBCORE

    def build_and_fire(g, slot, sem):
        a0 = s0 - 8 * g - 896
        fbases = []
        for r in range(8):
            ar = a0 - r
            sh = lax.bitwise_and(ar, NSHIFT - 1)
            fbases.append(sh * TABW + (ar - sh))

        @plsc.parallel_loop(0, BUILD_W // 128)
        def build_body(u):
            col = u * 128
            for r in range(8):
                for j in range(8):
                    off = col + 16 * j
                    v = tab16_v[pl.ds(pl.multiple_of(fbases[r] + off, 16), 16)]
                    b_v[slot, r, pl.ds(pl.multiple_of(off, 16), 16)] = v
        for n in range(8):
            i0 = i_base + 8 * g + 128 * n
            pltpu.make_async_copy(
                b_v.at[slot, :, pl.ds(pl.multiple_of(896 - 128 * n, 128), SEQ)],
                out_hbm.at[0, head, pl.ds(i0, 8), :],
                sem,
            ).start()

    def drain_group(slot, sem):
        for _ in range(8):
            pltpu.make_async_copy(
                b_v.at[slot, :, pl.ds(0, SEQ)],
                out_hbm.at[0, head, pl.ds(i_base, 8), :],
                sem,
            ).wait()

    def pair_body(gp, carry):
        @pl.when(gp > 0)
        def _():
            drain_group(0, sem0)

        build_and_fire(2 * gp, 0, sem0)

        @pl.when(gp > 0)
        def _():
            drain_group(1, sem1)

        build_and_fire(2 * gp + 1, 1, sem1)
        return carry

    lax.fori_loop(0, NGROUPS // 2, pair_body, 0)
    drain_group(0, sem0)
    drain_group(1, sem1)


@functools.cache
def _sc_expand():
    return pl.kernel(
        _sc_expand_body,
        out_type=jax.ShapeDtypeStruct((1, N_HEADS, SEQ, SEQ), jnp.float32),
        mesh=plsc.VectorSubcoreMesh(core_axis_name="c", subcore_axis_name="s"),
        scratch_types=[
            pltpu.VMEM((NSHIFT * TABW,), jnp.float32),
            pltpu.VMEM((2, 8, BW), jnp.float32),
            pltpu.SemaphoreType.DMA,
            pltpu.SemaphoreType.DMA,
        ],
    )


def kernel(relative_attention_bias, q_len, k_len, bidirectional):
    d = jnp.asarray(k_len, jnp.int32) - jnp.asarray(q_len, jnp.int32)
    d_arr = jnp.reshape(d, (1, 1))
    bid_arr = jnp.reshape(jnp.asarray(bidirectional, jnp.int32), (1, 1))
    bias_t = relative_attention_bias.T
    tables = _compute_tables(bias_t, d_arr, bid_arr)
    return _sc_expand()(jnp.reshape(tables, (N_HEADS * NSHIFT * TABW,)))
